# baseline (speedup 1.0000x reference)
import functools

import jax
import jax.numpy as jnp
from jax.experimental import pallas as pl
from jax.experimental.pallas import tpu as pltpu
from jax.experimental.pallas import tpu_sc as plsc

H = 16
S = 2048
D = 1024
HD = 64

BQ = 512
NQ = S // BQ
BV = 512
BM = 256

_SCALE = 1.0 / (D ** 0.5)


def _gelu(x):
    return x * 0.5 * (1.0 + jax.lax.erf(x * (2.0 ** -0.5)))


def _qkv_body(x_ref, w_ref, b_ref, o_ref):
    x = x_ref[...]
    w = w_ref[0]
    acc = jax.lax.dot_general(x, w, (((1,), (1,)), ((), ())),
                              preferred_element_type=jnp.float32)
    o_ref[0] = (acc + b_ref[0]).astype(jnp.bfloat16)


def _attn_body(q_ref, k_ref, v_ref, o_ref, gexp_ref, idx_ref):
    hh = pl.program_id(0)
    io = jax.lax.broadcasted_iota(jnp.int32, (S, 1), 0)
    hi = (io >> 8).astype(jnp.bfloat16)
    lo = (io & 255).astype(jnp.bfloat16)
    idx_rhs = jnp.concatenate([hi, lo], axis=1)
    ones_col = jnp.ones((S, 1), jnp.bfloat16)
    for j in range(2):
        q = q_ref[j]
        k = k_ref[j]
        v = v_ref[j]
        s = jax.lax.dot_general(q, k, (((1,), (1,)), ((), ())),
                                preferred_element_type=jnp.float32) * _SCALE
        m = jnp.max(s, axis=1, keepdims=True)
        p16 = jnp.exp(s - m).astype(jnp.bfloat16)
        oh16 = (s == m).astype(jnp.bfloat16)
        v_aug = jnp.concatenate([v, ones_col], axis=1)
        pv_aug = jax.lax.dot_general(p16, v_aug, (((1,), (0,)), ((), ())),
                                     preferred_element_type=jnp.float32)
        pv = pv_aug[:, :HD]
        l = pv_aug[:, HD:HD + 1]
        ohg = jax.lax.dot_general(oh16, idx_rhs, (((1,), (0,)), ((), ())),
                                  preferred_element_type=jnp.float32)
        idxf = ohg[:, 0:1] * 256.0 + ohg[:, 1:2]
        idxi = jnp.clip(idxf, 0.0, float(S - 1)).astype(jnp.int32)
        ginv = 1.0 / l
        sl = slice(j * HD, (j + 1) * HD)
        o_ref[:, sl] = (pv * ginv).astype(jnp.bfloat16)
        gexp_ref[:, sl] = jnp.broadcast_to(ginv.astype(jnp.bfloat16),
                                           (BQ, HD))
        idx_ref[0, 0, :, j:j + 1] = idxi + (2 * hh + j) * S


_NW = 32
_SC_ROWS = (S * H) // _NW

_sc_mesh = plsc.VectorSubcoreMesh(core_axis_name="c", subcore_axis_name="s")


@functools.partial(
    pl.kernel,
    mesh=_sc_mesh,
    compiler_params=pltpu.CompilerParams(use_tc_tiling_on_sc=False),
    out_type=jax.ShapeDtypeStruct((S * H, HD), jnp.bfloat16),
    scratch_types=[
        pltpu.VMEM((_SC_ROWS,), jnp.int32),
        pltpu.VMEM((_SC_ROWS, HD), jnp.bfloat16),
        pltpu.SemaphoreType.DMA,
    ],
)
def _sc_gather(table_hbm, idx_hbm, out_hbm, idx_v, rows_v, sem):
    wid = jax.lax.axis_index("s") * 2 + jax.lax.axis_index("c")
    base = wid * _SC_ROWS
    pltpu.sync_copy(idx_hbm.at[pl.ds(base, _SC_ROWS)], idx_v)
    pltpu.async_copy(table_hbm.at[idx_v], rows_v, sem).wait()
    pltpu.sync_copy(rows_v, out_hbm.at[pl.ds(base, _SC_ROWS)])


def _vb_body(g_ref, wb_ref, bb_ref, v_ref):
    g = g_ref[...]
    wb = wb_ref[...]
    acc = jax.lax.dot_general(g, wb, (((1,), (1,)), ((), ())),
                              preferred_element_type=jnp.float32)
    v_ref[...] = _gelu(acc + bb_ref[...])


def _ffn_body(o_ref, gexp_ref, vg_ref, v_ref, wf1_ref, bf1_ref,
              wf2_ref, bf2_ref, gam_ref, bet_ref, y_ref):
    vcol = v_ref[...]
    x0 = (o_ref[...].astype(jnp.float32)
          + (vcol - gexp_ref[...].astype(jnp.float32))
          * vg_ref[...].astype(jnp.float32))
    h1 = jax.lax.dot_general(x0.astype(jnp.bfloat16), wf1_ref[...],
                             (((1,), (1,)), ((), ())),
                             preferred_element_type=jnp.float32)
    h1 = _gelu(h1 + bf1_ref[...])
    h2 = jax.lax.dot_general(h1.astype(jnp.bfloat16), wf2_ref[...],
                             (((1,), (1,)), ((), ())),
                             preferred_element_type=jnp.float32)
    h2 = h2 + bf2_ref[...]
    mu = jnp.mean(h2, axis=1, keepdims=True)
    cen = h2 - mu
    var = jnp.mean(cen * cen, axis=1, keepdims=True)
    y_ref[...] = cen * jax.lax.rsqrt(var + 1e-5) * gam_ref[...] + bet_ref[...]


def kernel(x, W1, b1, W2, b2, W3, b3, Wb, bb, Wf1, bf1, Wf2, bf2,
           gamma, beta):
    xs = x.reshape(S, D).astype(jnp.bfloat16)
    w_all = (jnp.concatenate([W1, W2, W3], axis=0)
             .reshape(3 * H, HD, D).astype(jnp.bfloat16))
    b_all = jnp.concatenate([b1, b2, b3], axis=0).reshape(3 * H, 1, HD)

    qkv = pl.pallas_call(
        _qkv_body,
        grid=(3 * H,),
        in_specs=[
            pl.BlockSpec((S, D), lambda j: (0, 0)),
            pl.BlockSpec((1, HD, D), lambda j: (j, 0, 0)),
            pl.BlockSpec((1, 1, HD), lambda j: (j, 0, 0)),
        ],
        out_specs=pl.BlockSpec((1, S, HD), lambda j: (j, 0, 0)),
        out_shape=jax.ShapeDtypeStruct((3 * H, S, HD), jnp.bfloat16),
    )(xs, w_all, b_all)

    Q, K, V = qkv[:H], qkv[H:2 * H], qkv[2 * H:]

    O, Gexp, idxg = pl.pallas_call(
        _attn_body,
        grid=(H // 2, NQ),
        in_specs=[
            pl.BlockSpec((2, BQ, HD), lambda hh, q: (hh, q, 0)),
            pl.BlockSpec((2, S, HD), lambda hh, q: (hh, 0, 0)),
            pl.BlockSpec((2, S, HD), lambda hh, q: (hh, 0, 0)),
        ],
        out_specs=[
            pl.BlockSpec((BQ, 2 * HD), lambda hh, q: (q, hh)),
            pl.BlockSpec((BQ, 2 * HD), lambda hh, q: (q, hh)),
            pl.BlockSpec((1, 1, BQ, 2), lambda hh, q: (hh, q, 0, 0)),
        ],
        out_shape=[
            jax.ShapeDtypeStruct((S, D), jnp.bfloat16),
            jax.ShapeDtypeStruct((S, D), jnp.bfloat16),
            jax.ShapeDtypeStruct((H // 2, NQ, BQ, 2), jnp.int32),
        ],
    )(Q, K, V)

    idx_t = idxg.transpose(1, 2, 0, 3).reshape(S * H)
    Vg = _sc_gather(V.reshape(H * S, HD), idx_t).reshape(S, D)

    g_row = jax.lax.slice(Gexp, (0, (H - 1) * HD), (S, (H - 1) * HD + 1))
    g_row = g_row.reshape(1, S)

    v_vec = pl.pallas_call(
        _vb_body,
        grid=(S // BV,),
        in_specs=[
            pl.BlockSpec((1, S), lambda i: (0, 0)),
            pl.BlockSpec((BV, S), lambda i: (i, 0)),
            pl.BlockSpec((1, BV), lambda i: (0, i)),
        ],
        out_specs=pl.BlockSpec((1, BV), lambda i: (0, i)),
        out_shape=jax.ShapeDtypeStruct((1, S), jnp.float32),
    )(g_row, Wb.astype(jnp.bfloat16), bb.reshape(1, S))
    v_col = v_vec.reshape(S, 1)

    y = pl.pallas_call(
        _ffn_body,
        grid=(S // BM,),
        in_specs=[
            pl.BlockSpec((BM, D), lambda i: (i, 0)),
            pl.BlockSpec((BM, D), lambda i: (i, 0)),
            pl.BlockSpec((BM, D), lambda i: (i, 0)),
            pl.BlockSpec((BM, 1), lambda i: (i, 0)),
            pl.BlockSpec((4 * D, D), lambda i: (0, 0)),
            pl.BlockSpec((1, 4 * D), lambda i: (0, 0)),
            pl.BlockSpec((D, 4 * D), lambda i: (0, 0)),
            pl.BlockSpec((1, D), lambda i: (0, 0)),
            pl.BlockSpec((1, D), lambda i: (0, 0)),
            pl.BlockSpec((1, D), lambda i: (0, 0)),
        ],
        out_specs=pl.BlockSpec((BM, D), lambda i: (i, 0)),
        out_shape=jax.ShapeDtypeStruct((S, D), jnp.float32),
    )(O, Gexp, Vg, v_col, Wf1.astype(jnp.bfloat16),
      bf1.reshape(1, 4 * D), Wf2.astype(jnp.bfloat16),
      bf2.reshape(1, D), gamma.reshape(1, D), beta.reshape(1, D))

    return y.reshape(1, S, D)


# speedup vs baseline: 1.1180x; 1.1180x over previous
import functools

import jax
import jax.numpy as jnp
from jax.experimental import pallas as pl
from jax.experimental.pallas import tpu as pltpu
from jax.experimental.pallas import tpu_sc as plsc

H = 16
S = 2048
D = 1024
HD = 64

BQ = 512
NQ = S // BQ
BV = 512
BM = 256

_SCALE = 1.0 / (D ** 0.5)


def _gelu(x):
    return x * 0.5 * (1.0 + jax.lax.erf(x * (2.0 ** -0.5)))


def _qkv_body(x_ref, w1_ref, w2_ref, w3_ref, b_ref, o_ref):
    j = pl.program_id(0)
    x = x_ref[...]
    w = jnp.where(j < H, w1_ref[0],
                  jnp.where(j < 2 * H, w2_ref[0], w3_ref[0]))
    acc = jax.lax.dot_general(x, w.astype(jnp.bfloat16),
                              (((1,), (1,)), ((), ())),
                              preferred_element_type=jnp.float32)
    o_ref[0] = (acc + b_ref[0]).astype(jnp.bfloat16)


def _attn_body(q_ref, k_ref, v_ref, o_ref, gexp_ref, idx_ref):
    hh = pl.program_id(0)
    io = jax.lax.broadcasted_iota(jnp.int32, (S, 1), 0)
    hi = (io >> 8).astype(jnp.bfloat16)
    lo = (io & 255).astype(jnp.bfloat16)
    idx_rhs = jnp.concatenate([hi, lo], axis=1)
    ones_col = jnp.ones((S, 1), jnp.bfloat16)
    for j in range(2):
        q = q_ref[j]
        k = k_ref[j]
        v = v_ref[j]
        s = jax.lax.dot_general(q, k, (((1,), (1,)), ((), ())),
                                preferred_element_type=jnp.float32) * _SCALE
        m = jnp.max(s, axis=1, keepdims=True)
        p16 = jnp.exp(s - m).astype(jnp.bfloat16)
        oh16 = (s == m).astype(jnp.bfloat16)
        v_aug = jnp.concatenate([v, ones_col], axis=1)
        pv_aug = jax.lax.dot_general(p16, v_aug, (((1,), (0,)), ((), ())),
                                     preferred_element_type=jnp.float32)
        pv = pv_aug[:, :HD]
        l = pv_aug[:, HD:HD + 1]
        ohg = jax.lax.dot_general(oh16, idx_rhs, (((1,), (0,)), ((), ())),
                                  preferred_element_type=jnp.float32)
        idxf = ohg[:, 0:1] * 256.0 + ohg[:, 1:2]
        idxi = jnp.clip(idxf, 0.0, float(S - 1)).astype(jnp.int32)
        ginv = 1.0 / l
        sl = slice(j * HD, (j + 1) * HD)
        o_ref[:, sl] = (pv * ginv).astype(jnp.bfloat16)
        gexp_ref[:, sl] = jnp.broadcast_to(ginv.astype(jnp.bfloat16),
                                           (BQ, HD))
        idx_ref[0, 0, :, j:j + 1] = idxi + (2 * hh + j) * S


_NW = 32
_SC_ROWS = (S * H) // _NW

_sc_mesh = plsc.VectorSubcoreMesh(core_axis_name="c", subcore_axis_name="s")


@functools.partial(
    pl.kernel,
    mesh=_sc_mesh,
    compiler_params=pltpu.CompilerParams(use_tc_tiling_on_sc=False),
    out_type=jax.ShapeDtypeStruct((S * H, HD), jnp.bfloat16),
    scratch_types=[
        pltpu.VMEM((_SC_ROWS,), jnp.int32),
        pltpu.VMEM((_SC_ROWS, HD), jnp.bfloat16),
        pltpu.SemaphoreType.DMA,
    ],
)
def _sc_gather(table_hbm, idx_hbm, out_hbm, idx_v, rows_v, sem):
    wid = jax.lax.axis_index("s") * 2 + jax.lax.axis_index("c")
    base = wid * _SC_ROWS
    pltpu.sync_copy(idx_hbm.at[pl.ds(base, _SC_ROWS)], idx_v)
    pltpu.async_copy(table_hbm.at[idx_v], rows_v, sem).wait()
    pltpu.sync_copy(rows_v, out_hbm.at[pl.ds(base, _SC_ROWS)])


def _vb_body(g_ref, wb_ref, bb_ref, v_ref):
    g = g_ref[...]
    wb = wb_ref[...].astype(jnp.bfloat16)
    acc = jax.lax.dot_general(g, wb, (((1,), (1,)), ((), ())),
                              preferred_element_type=jnp.float32)
    v_ref[...] = _gelu(acc + bb_ref[...])


def _ffn_body(o_ref, gexp_ref, vg_ref, v_ref, wf1_ref, bf1_ref,
              wf2_ref, bf2_ref, gam_ref, bet_ref, y_ref):
    vcol = v_ref[...]
    x0 = (o_ref[...].astype(jnp.float32)
          + (vcol - gexp_ref[...].astype(jnp.float32))
          * vg_ref[...].astype(jnp.float32))
    h1 = jax.lax.dot_general(x0.astype(jnp.bfloat16),
                             wf1_ref[...].astype(jnp.bfloat16),
                             (((1,), (1,)), ((), ())),
                             preferred_element_type=jnp.float32)
    h1 = _gelu(h1 + bf1_ref[...])
    h2 = jax.lax.dot_general(h1.astype(jnp.bfloat16),
                             wf2_ref[...].astype(jnp.bfloat16),
                             (((1,), (1,)), ((), ())),
                             preferred_element_type=jnp.float32)
    h2 = h2 + bf2_ref[...]
    mu = jnp.mean(h2, axis=1, keepdims=True)
    cen = h2 - mu
    var = jnp.mean(cen * cen, axis=1, keepdims=True)
    y_ref[...] = cen * jax.lax.rsqrt(var + 1e-5) * gam_ref[...] + bet_ref[...]


def kernel(x, W1, b1, W2, b2, W3, b3, Wb, bb, Wf1, bf1, Wf2, bf2,
           gamma, beta):
    xs = x.reshape(S, D).astype(jnp.bfloat16)
    w1r = W1.reshape(H, HD, D)
    w2r = W2.reshape(H, HD, D)
    w3r = W3.reshape(H, HD, D)
    b_all = jnp.concatenate([b1, b2, b3], axis=0).reshape(3 * H, 1, HD)

    qkv = pl.pallas_call(
        _qkv_body,
        grid=(3 * H,),
        in_specs=[
            pl.BlockSpec((S, D), lambda j: (0, 0)),
            pl.BlockSpec((1, HD, D),
                         lambda j: (jnp.where(j < H, j, 0), 0, 0)),
            pl.BlockSpec((1, HD, D),
                         lambda j: (jnp.where((j >= H) & (j < 2 * H),
                                              j - H, 0), 0, 0)),
            pl.BlockSpec((1, HD, D),
                         lambda j: (jnp.where(j >= 2 * H, j - 2 * H, 0),
                                    0, 0)),
            pl.BlockSpec((1, 1, HD), lambda j: (j, 0, 0)),
        ],
        out_specs=pl.BlockSpec((1, S, HD), lambda j: (j, 0, 0)),
        out_shape=jax.ShapeDtypeStruct((3 * H, S, HD), jnp.bfloat16),
    )(xs, w1r, w2r, w3r, b_all)

    V = qkv[2 * H:]

    O, Gexp, idxg = pl.pallas_call(
        _attn_body,
        grid=(H // 2, NQ),
        in_specs=[
            pl.BlockSpec((2, BQ, HD), lambda hh, q: (hh, q, 0)),
            pl.BlockSpec((2, S, HD), lambda hh, q: (8 + hh, 0, 0)),
            pl.BlockSpec((2, S, HD), lambda hh, q: (16 + hh, 0, 0)),
        ],
        out_specs=[
            pl.BlockSpec((BQ, 2 * HD), lambda hh, q: (q, hh)),
            pl.BlockSpec((BQ, 2 * HD), lambda hh, q: (q, hh)),
            pl.BlockSpec((1, 1, BQ, 2), lambda hh, q: (hh, q, 0, 0)),
        ],
        out_shape=[
            jax.ShapeDtypeStruct((S, D), jnp.bfloat16),
            jax.ShapeDtypeStruct((S, D), jnp.bfloat16),
            jax.ShapeDtypeStruct((H // 2, NQ, BQ, 2), jnp.int32),
        ],
    )(qkv, qkv, qkv)

    idx_t = idxg.transpose(1, 2, 0, 3).reshape(S * H)
    Vg = _sc_gather(V.reshape(H * S, HD), idx_t).reshape(S, D)

    g_row = jax.lax.slice(Gexp, (0, (H - 1) * HD), (S, (H - 1) * HD + 1))
    g_row = g_row.reshape(1, S)

    v_vec = pl.pallas_call(
        _vb_body,
        grid=(S // BV,),
        in_specs=[
            pl.BlockSpec((1, S), lambda i: (0, 0)),
            pl.BlockSpec((BV, S), lambda i: (i, 0)),
            pl.BlockSpec((1, BV), lambda i: (0, i)),
        ],
        out_specs=pl.BlockSpec((1, BV), lambda i: (0, i)),
        out_shape=jax.ShapeDtypeStruct((1, S), jnp.float32),
    )(g_row, Wb, bb.reshape(1, S))
    v_col = v_vec.reshape(S, 1)

    y = pl.pallas_call(
        _ffn_body,
        grid=(S // BM,),
        in_specs=[
            pl.BlockSpec((BM, D), lambda i: (i, 0)),
            pl.BlockSpec((BM, D), lambda i: (i, 0)),
            pl.BlockSpec((BM, D), lambda i: (i, 0)),
            pl.BlockSpec((BM, 1), lambda i: (i, 0)),
            pl.BlockSpec((4 * D, D), lambda i: (0, 0)),
            pl.BlockSpec((1, 4 * D), lambda i: (0, 0)),
            pl.BlockSpec((D, 4 * D), lambda i: (0, 0)),
            pl.BlockSpec((1, D), lambda i: (0, 0)),
            pl.BlockSpec((1, D), lambda i: (0, 0)),
            pl.BlockSpec((1, D), lambda i: (0, 0)),
        ],
        out_specs=pl.BlockSpec((BM, D), lambda i: (i, 0)),
        out_shape=jax.ShapeDtypeStruct((S, D), jnp.float32),
    )(O, Gexp, Vg, v_col, Wf1, bf1.reshape(1, 4 * D), Wf2,
      bf2.reshape(1, D), gamma.reshape(1, D), beta.reshape(1, D))

    return y.reshape(1, S, D)


# speedup vs baseline: 1.1466x; 1.0255x over previous
import functools

import jax
import jax.numpy as jnp
from jax.experimental import pallas as pl
from jax.experimental.pallas import tpu as pltpu
from jax.experimental.pallas import tpu_sc as plsc

H = 16
S = 2048
D = 1024
HD = 64

BQ = 1024
NQ = S // BQ
BV = 512
BM = 512

_SCALE = 1.0 / (D ** 0.5)


def _gelu(x):
    return x * 0.5 * (1.0 + jax.lax.erf(x * (2.0 ** -0.5)))


def _qkv_body(x_ref, w1_ref, w2_ref, w3_ref, b_ref, o_ref):
    j = pl.program_id(0)
    x = x_ref[...]
    w = jnp.where(j < H, w1_ref[0],
                  jnp.where(j < 2 * H, w2_ref[0], w3_ref[0]))
    acc = jax.lax.dot_general(x, w.astype(jnp.bfloat16),
                              (((1,), (1,)), ((), ())),
                              preferred_element_type=jnp.float32)
    o_ref[0] = (acc + b_ref[0]).astype(jnp.bfloat16)


def _attn_body(q_ref, k_ref, v_ref, o_ref, gexp_ref, idx_ref):
    hh = pl.program_id(0)
    io = jax.lax.broadcasted_iota(jnp.int32, (S, 1), 0)
    hi = (io >> 8).astype(jnp.bfloat16)
    lo = (io & 255).astype(jnp.bfloat16)
    idx_rhs = jnp.concatenate([hi, lo], axis=1)
    ones_col = jnp.ones((S, 1), jnp.bfloat16)
    for j in range(2):
        q = q_ref[j]
        k = k_ref[j]
        v = v_ref[j]
        s = jax.lax.dot_general(q, k, (((1,), (1,)), ((), ())),
                                preferred_element_type=jnp.float32) * _SCALE
        m = jnp.max(s, axis=1, keepdims=True)
        p16 = jnp.exp(s - m).astype(jnp.bfloat16)
        oh16 = (s == m).astype(jnp.bfloat16)
        v_aug = jnp.concatenate([v, ones_col], axis=1)
        pv_aug = jax.lax.dot_general(p16, v_aug, (((1,), (0,)), ((), ())),
                                     preferred_element_type=jnp.float32)
        pv = pv_aug[:, :HD]
        l = pv_aug[:, HD:HD + 1]
        ohg = jax.lax.dot_general(oh16, idx_rhs, (((1,), (0,)), ((), ())),
                                  preferred_element_type=jnp.float32)
        idxf = ohg[:, 0:1] * 256.0 + ohg[:, 1:2]
        idxi = jnp.clip(idxf, 0.0, float(S - 1)).astype(jnp.int32)
        ginv = 1.0 / l
        sl = slice(j * HD, (j + 1) * HD)
        o_ref[:, sl] = (pv * ginv).astype(jnp.bfloat16)
        gexp_ref[:, sl] = jnp.broadcast_to(ginv.astype(jnp.bfloat16),
                                           (BQ, HD))
        idx_ref[0, 0, :, j:j + 1] = idxi + (2 * hh + j) * S


_NW = 32
_SC_ROWS = (S * H) // _NW

_sc_mesh = plsc.VectorSubcoreMesh(core_axis_name="c", subcore_axis_name="s")


@functools.partial(
    pl.kernel,
    mesh=_sc_mesh,
    compiler_params=pltpu.CompilerParams(use_tc_tiling_on_sc=False),
    out_type=jax.ShapeDtypeStruct((S * H, HD), jnp.bfloat16),
    scratch_types=[
        pltpu.VMEM((_SC_ROWS,), jnp.int32),
        pltpu.VMEM((_SC_ROWS, HD), jnp.bfloat16),
        pltpu.SemaphoreType.DMA,
    ],
)
def _sc_gather(table_hbm, idx_hbm, out_hbm, idx_v, rows_v, sem):
    wid = jax.lax.axis_index("s") * 2 + jax.lax.axis_index("c")
    base = wid * _SC_ROWS
    pltpu.sync_copy(idx_hbm.at[pl.ds(base, _SC_ROWS)], idx_v)
    pltpu.async_copy(table_hbm.at[idx_v], rows_v, sem).wait()
    pltpu.sync_copy(rows_v, out_hbm.at[pl.ds(base, _SC_ROWS)])


def _vb_body(g_ref, wb_ref, bb_ref, v_ref):
    g = g_ref[...]
    wb = wb_ref[...].astype(jnp.bfloat16)
    acc = jax.lax.dot_general(g, wb, (((1,), (1,)), ((), ())),
                              preferred_element_type=jnp.float32)
    v_ref[...] = _gelu(acc + bb_ref[...])


def _ffn_body(o_ref, gexp_ref, vg_ref, v_ref, wf1_ref, bf1_ref,
              wf2_ref, bf2_ref, gam_ref, bet_ref, y_ref):
    vcol = v_ref[...]
    x0 = (o_ref[...].astype(jnp.float32)
          + (vcol - gexp_ref[...].astype(jnp.float32))
          * vg_ref[...].astype(jnp.float32))
    h1 = jax.lax.dot_general(x0.astype(jnp.bfloat16),
                             wf1_ref[...].astype(jnp.bfloat16),
                             (((1,), (1,)), ((), ())),
                             preferred_element_type=jnp.float32)
    h1 = _gelu(h1 + bf1_ref[...])
    h2 = jax.lax.dot_general(h1.astype(jnp.bfloat16),
                             wf2_ref[...].astype(jnp.bfloat16),
                             (((1,), (1,)), ((), ())),
                             preferred_element_type=jnp.float32)
    h2 = h2 + bf2_ref[...]
    mu = jnp.mean(h2, axis=1, keepdims=True)
    cen = h2 - mu
    var = jnp.mean(cen * cen, axis=1, keepdims=True)
    y_ref[...] = cen * jax.lax.rsqrt(var + 1e-5) * gam_ref[...] + bet_ref[...]


def kernel(x, W1, b1, W2, b2, W3, b3, Wb, bb, Wf1, bf1, Wf2, bf2,
           gamma, beta):
    xs = x.reshape(S, D).astype(jnp.bfloat16)
    w1r = W1.reshape(H, HD, D)
    w2r = W2.reshape(H, HD, D)
    w3r = W3.reshape(H, HD, D)
    b_all = jnp.concatenate([b1, b2, b3], axis=0).reshape(3 * H, 1, HD)

    qkv = pl.pallas_call(
        _qkv_body,
        grid=(3 * H,),
        in_specs=[
            pl.BlockSpec((S, D), lambda j: (0, 0)),
            pl.BlockSpec((1, HD, D),
                         lambda j: (jnp.where(j < H, j, 0), 0, 0)),
            pl.BlockSpec((1, HD, D),
                         lambda j: (jnp.where((j >= H) & (j < 2 * H),
                                              j - H, 0), 0, 0)),
            pl.BlockSpec((1, HD, D),
                         lambda j: (jnp.where(j >= 2 * H, j - 2 * H, 0),
                                    0, 0)),
            pl.BlockSpec((1, 1, HD), lambda j: (j, 0, 0)),
        ],
        out_specs=pl.BlockSpec((1, S, HD), lambda j: (j, 0, 0)),
        out_shape=jax.ShapeDtypeStruct((3 * H, S, HD), jnp.bfloat16),
    )(xs, w1r, w2r, w3r, b_all)

    V = qkv[2 * H:]

    O, Gexp, idxg = pl.pallas_call(
        _attn_body,
        grid=(H // 2, NQ),
        in_specs=[
            pl.BlockSpec((2, BQ, HD), lambda hh, q: (hh, q, 0)),
            pl.BlockSpec((2, S, HD), lambda hh, q: (8 + hh, 0, 0)),
            pl.BlockSpec((2, S, HD), lambda hh, q: (16 + hh, 0, 0)),
        ],
        out_specs=[
            pl.BlockSpec((BQ, 2 * HD), lambda hh, q: (q, hh)),
            pl.BlockSpec((BQ, 2 * HD), lambda hh, q: (q, hh)),
            pl.BlockSpec((1, 1, BQ, 2), lambda hh, q: (hh, q, 0, 0)),
        ],
        out_shape=[
            jax.ShapeDtypeStruct((S, D), jnp.bfloat16),
            jax.ShapeDtypeStruct((S, D), jnp.bfloat16),
            jax.ShapeDtypeStruct((H // 2, NQ, BQ, 2), jnp.int32),
        ],
    )(qkv, qkv, qkv)

    idx_t = idxg.transpose(1, 2, 0, 3).reshape(S * H)
    Vg = _sc_gather(V.reshape(H * S, HD), idx_t).reshape(S, D)

    g_row = jax.lax.slice(Gexp, (0, (H - 1) * HD), (S, (H - 1) * HD + 1))
    g_row = g_row.reshape(1, S)

    v_vec = pl.pallas_call(
        _vb_body,
        grid=(S // BV,),
        in_specs=[
            pl.BlockSpec((1, S), lambda i: (0, 0)),
            pl.BlockSpec((BV, S), lambda i: (i, 0)),
            pl.BlockSpec((1, BV), lambda i: (0, i)),
        ],
        out_specs=pl.BlockSpec((1, BV), lambda i: (0, i)),
        out_shape=jax.ShapeDtypeStruct((1, S), jnp.float32),
    )(g_row, Wb, bb.reshape(1, S))
    v_col = v_vec.reshape(S, 1)

    y = pl.pallas_call(
        _ffn_body,
        grid=(S // BM,),
        in_specs=[
            pl.BlockSpec((BM, D), lambda i: (i, 0)),
            pl.BlockSpec((BM, D), lambda i: (i, 0)),
            pl.BlockSpec((BM, D), lambda i: (i, 0)),
            pl.BlockSpec((BM, 1), lambda i: (i, 0)),
            pl.BlockSpec((4 * D, D), lambda i: (0, 0)),
            pl.BlockSpec((1, 4 * D), lambda i: (0, 0)),
            pl.BlockSpec((D, 4 * D), lambda i: (0, 0)),
            pl.BlockSpec((1, D), lambda i: (0, 0)),
            pl.BlockSpec((1, D), lambda i: (0, 0)),
            pl.BlockSpec((1, D), lambda i: (0, 0)),
        ],
        out_specs=pl.BlockSpec((BM, D), lambda i: (i, 0)),
        out_shape=jax.ShapeDtypeStruct((S, D), jnp.float32),
    )(O, Gexp, Vg, v_col, Wf1, bf1.reshape(1, 4 * D), Wf2,
      bf2.reshape(1, D), gamma.reshape(1, D), beta.reshape(1, D))

    return y.reshape(1, S, D)


# speedup vs baseline: 1.4703x; 1.2823x over previous
import functools

import jax
import jax.numpy as jnp
from jax.experimental import pallas as pl
from jax.experimental.pallas import tpu as pltpu
from jax.experimental.pallas import tpu_sc as plsc

H = 16
S = 2048
D = 1024
HD = 64

BQ = 1024
NQ = S // BQ
BV = 512
BM = 512

_SCALE = 1.0 / (D ** 0.5)


def _gelu(x):
    return x * 0.5 * (1.0 + jax.lax.erf(x * (2.0 ** -0.5)))


def _qkv_body(x_ref, w1_ref, w2_ref, w3_ref, b_ref, o_ref):
    j = pl.program_id(0)
    x = x_ref[...]
    w = jnp.where(j < 4, w1_ref[0],
                  jnp.where(j < 8, w2_ref[0], w3_ref[0]))
    acc = jax.lax.dot_general(x, w.astype(jnp.bfloat16),
                              (((1,), (1,)), ((), ())),
                              preferred_element_type=jnp.float32)
    o_ref[0] = (acc + b_ref[0]).astype(jnp.bfloat16)


def _attn_body(q_ref, k_ref, v_ref, o_ref, gexp_ref, idx_ref):
    g4 = pl.program_id(0)
    io = jax.lax.broadcasted_iota(jnp.int32, (S, 1), 0)
    hi = (io >> 8).astype(jnp.bfloat16)
    lo = (io & 255).astype(jnp.bfloat16)
    idx_rhs = jnp.concatenate([hi, lo], axis=1)
    ones_col = jnp.ones((S, 1), jnp.bfloat16)
    for j in range(4):
        sl = slice(j * HD, (j + 1) * HD)
        q = q_ref[0][:, sl]
        k = k_ref[0][:, sl]
        v = v_ref[0][:, sl]
        s = jax.lax.dot_general(q, k, (((1,), (1,)), ((), ())),
                                preferred_element_type=jnp.float32) * _SCALE
        m = jnp.max(s, axis=1, keepdims=True)
        p16 = jnp.exp(s - m).astype(jnp.bfloat16)
        oh16 = (s == m).astype(jnp.bfloat16)
        v_aug = jnp.concatenate([v, ones_col], axis=1)
        pv_aug = jax.lax.dot_general(p16, v_aug, (((1,), (0,)), ((), ())),
                                     preferred_element_type=jnp.float32)
        pv = pv_aug[:, :HD]
        l = pv_aug[:, HD:HD + 1]
        ohg = jax.lax.dot_general(oh16, idx_rhs, (((1,), (0,)), ((), ())),
                                  preferred_element_type=jnp.float32)
        idxf = ohg[:, 0:1] * 256.0 + ohg[:, 1:2]
        idxi = jnp.clip(idxf, 0.0, float(S - 1)).astype(jnp.int32)
        ginv = 1.0 / l
        o_ref[:, sl] = (pv * ginv).astype(jnp.bfloat16)
        gexp_ref[:, sl] = jnp.broadcast_to(ginv.astype(jnp.bfloat16),
                                           (BQ, HD))
        idx_ref[0, 0, :, j:j + 1] = (idxi + g4 * S) * 4 + j


_NW = 32
_SC_ROWS = (S * H) // _NW

_sc_mesh = plsc.VectorSubcoreMesh(core_axis_name="c", subcore_axis_name="s")


@functools.partial(
    pl.kernel,
    mesh=_sc_mesh,
    compiler_params=pltpu.CompilerParams(use_tc_tiling_on_sc=False),
    out_type=jax.ShapeDtypeStruct((S * H, HD), jnp.bfloat16),
    scratch_types=[
        pltpu.VMEM((_SC_ROWS,), jnp.int32),
        pltpu.VMEM((_SC_ROWS, HD), jnp.bfloat16),
        pltpu.SemaphoreType.DMA,
    ],
)
def _sc_gather(table_hbm, idx_hbm, out_hbm, idx_v, rows_v, sem):
    wid = jax.lax.axis_index("s") * 2 + jax.lax.axis_index("c")
    base = wid * _SC_ROWS
    pltpu.sync_copy(idx_hbm.at[pl.ds(base, _SC_ROWS)], idx_v)
    pltpu.async_copy(table_hbm.at[idx_v], rows_v, sem).wait()
    pltpu.sync_copy(rows_v, out_hbm.at[pl.ds(base, _SC_ROWS)])


def _vb_body(g_ref, wb_ref, bb_ref, v_ref):
    g = g_ref[...]
    wb = wb_ref[...].astype(jnp.bfloat16)
    acc = jax.lax.dot_general(g, wb, (((1,), (1,)), ((), ())),
                              preferred_element_type=jnp.float32)
    v_ref[...] = _gelu(acc + bb_ref[...])


def _ffn_body(o_ref, gexp_ref, vg_ref, v_ref, wf1_ref, bf1_ref,
              wf2_ref, bf2_ref, gam_ref, bet_ref, y_ref):
    vcol = v_ref[...]
    x0 = (o_ref[...].astype(jnp.float32)
          + (vcol - gexp_ref[...].astype(jnp.float32))
          * vg_ref[...].astype(jnp.float32))
    h1 = jax.lax.dot_general(x0.astype(jnp.bfloat16),
                             wf1_ref[...].astype(jnp.bfloat16),
                             (((1,), (1,)), ((), ())),
                             preferred_element_type=jnp.float32)
    h1 = _gelu(h1 + bf1_ref[...])
    h2 = jax.lax.dot_general(h1.astype(jnp.bfloat16),
                             wf2_ref[...].astype(jnp.bfloat16),
                             (((1,), (1,)), ((), ())),
                             preferred_element_type=jnp.float32)
    h2 = h2 + bf2_ref[...]
    mu = jnp.mean(h2, axis=1, keepdims=True)
    cen = h2 - mu
    var = jnp.mean(cen * cen, axis=1, keepdims=True)
    y_ref[...] = cen * jax.lax.rsqrt(var + 1e-5) * gam_ref[...] + bet_ref[...]


def kernel(x, W1, b1, W2, b2, W3, b3, Wb, bb, Wf1, bf1, Wf2, bf2,
           gamma, beta):
    xs = x.reshape(S, D).astype(jnp.bfloat16)
    w1r = W1.reshape(4, 4 * HD, D)
    w2r = W2.reshape(4, 4 * HD, D)
    w3r = W3.reshape(4, 4 * HD, D)
    b_all = jnp.concatenate([b1, b2, b3], axis=0).reshape(12, 1, 4 * HD)

    qkv = pl.pallas_call(
        _qkv_body,
        grid=(12,),
        in_specs=[
            pl.BlockSpec((S, D), lambda j: (0, 0)),
            pl.BlockSpec((1, 4 * HD, D),
                         lambda j: (jnp.where(j < 4, j, 0), 0, 0)),
            pl.BlockSpec((1, 4 * HD, D),
                         lambda j: (jnp.where((j >= 4) & (j < 8),
                                              j - 4, 0), 0, 0)),
            pl.BlockSpec((1, 4 * HD, D),
                         lambda j: (jnp.where(j >= 8, j - 8, 0), 0, 0)),
            pl.BlockSpec((1, 1, 4 * HD), lambda j: (j, 0, 0)),
        ],
        out_specs=pl.BlockSpec((1, S, 4 * HD), lambda j: (j, 0, 0)),
        out_shape=jax.ShapeDtypeStruct((12, S, 4 * HD), jnp.bfloat16),
    )(xs, w1r, w2r, w3r, b_all)

    V = qkv[8:]

    O, Gexp, idxg = pl.pallas_call(
        _attn_body,
        grid=(H // 4, NQ),
        in_specs=[
            pl.BlockSpec((1, BQ, 4 * HD), lambda g4, q: (g4, q, 0)),
            pl.BlockSpec((1, S, 4 * HD), lambda g4, q: (4 + g4, 0, 0)),
            pl.BlockSpec((1, S, 4 * HD), lambda g4, q: (8 + g4, 0, 0)),
        ],
        out_specs=[
            pl.BlockSpec((BQ, 4 * HD), lambda g4, q: (q, g4)),
            pl.BlockSpec((BQ, 4 * HD), lambda g4, q: (q, g4)),
            pl.BlockSpec((1, 1, BQ, 4), lambda g4, q: (g4, q, 0, 0)),
        ],
        out_shape=[
            jax.ShapeDtypeStruct((S, D), jnp.bfloat16),
            jax.ShapeDtypeStruct((S, D), jnp.bfloat16),
            jax.ShapeDtypeStruct((H // 4, NQ, BQ, 4), jnp.int32),
        ],
    )(qkv, qkv, qkv)

    idx_t = idxg.transpose(1, 2, 0, 3).reshape(S * H)
    Vg = _sc_gather(V.reshape(H * S, HD), idx_t).reshape(S, D)

    g_row = jax.lax.slice(Gexp, (0, (H - 1) * HD), (S, (H - 1) * HD + 1))
    g_row = g_row.reshape(1, S)

    v_vec = pl.pallas_call(
        _vb_body,
        grid=(S // BV,),
        in_specs=[
            pl.BlockSpec((1, S), lambda i: (0, 0)),
            pl.BlockSpec((BV, S), lambda i: (i, 0)),
            pl.BlockSpec((1, BV), lambda i: (0, i)),
        ],
        out_specs=pl.BlockSpec((1, BV), lambda i: (0, i)),
        out_shape=jax.ShapeDtypeStruct((1, S), jnp.float32),
    )(g_row, Wb, bb.reshape(1, S))
    v_col = v_vec.reshape(S, 1)

    y = pl.pallas_call(
        _ffn_body,
        grid=(S // BM,),
        in_specs=[
            pl.BlockSpec((BM, D), lambda i: (i, 0)),
            pl.BlockSpec((BM, D), lambda i: (i, 0)),
            pl.BlockSpec((BM, D), lambda i: (i, 0)),
            pl.BlockSpec((BM, 1), lambda i: (i, 0)),
            pl.BlockSpec((4 * D, D), lambda i: (0, 0)),
            pl.BlockSpec((1, 4 * D), lambda i: (0, 0)),
            pl.BlockSpec((D, 4 * D), lambda i: (0, 0)),
            pl.BlockSpec((1, D), lambda i: (0, 0)),
            pl.BlockSpec((1, D), lambda i: (0, 0)),
            pl.BlockSpec((1, D), lambda i: (0, 0)),
        ],
        out_specs=pl.BlockSpec((BM, D), lambda i: (i, 0)),
        out_shape=jax.ShapeDtypeStruct((S, D), jnp.float32),
    )(O, Gexp, Vg, v_col, Wf1, bf1.reshape(1, 4 * D), Wf2,
      bf2.reshape(1, D), gamma.reshape(1, D), beta.reshape(1, D))

    return y.reshape(1, S, D)
